# baseline (speedup 1.0000x reference)
import functools

import jax
import jax.numpy as jnp
from jax import lax
from jax.experimental import pallas as pl
from jax.experimental.pallas import tpu as pltpu
from jax.experimental.pallas import tpu_sc as plsc

_N = 10000
_NP = 10240
_H = 256
_HH = 128
_E = 160000
_K = 20
_NM = 8
_NTILES = 16
_NCORES = 2
_CHUNK = 128
_NCHUNK = 80
_EPT = _NCHUNK * _CHUNK
_EP = _EPT * _NTILES
_T32 = _EP // 32
_DUMMY = _N
_BM = 1024
_GRID = _NP // _BM


def _deg_body(src_hbm, dst_hbm, out_do, out_di, src_v, dst_v, dego_v, degi_v):
    c = lax.axis_index("c")
    s = lax.axis_index("s")
    wid = s * _NCORES + c
    zero16 = jnp.zeros((16,), jnp.float32)

    def zloop(i, carry):
        dego_v[pl.ds(i * 16, 16)] = zero16
        degi_v[pl.ds(i * 16, 16)] = zero16
        return carry

    lax.fori_loop(0, _NP // 16, zloop, 0)

    base = wid * _T32
    pltpu.sync_copy(src_hbm.at[pl.ds(base, _T32)], src_v)
    pltpu.sync_copy(dst_hbm.at[pl.ds(base, _T32)], dst_v)
    ones16 = jnp.ones((16,), jnp.float32)

    def eloop(i, carry):
        si = src_v[pl.ds(i * 16, 16)]
        plsc.addupdate_scatter(dego_v, [si], ones16)
        di = dst_v[pl.ds(i * 16, 16)]
        plsc.addupdate_scatter(degi_v, [di], ones16)
        return carry

    lax.fori_loop(0, _T32 // 16, eloop, 0)
    pltpu.sync_copy(dego_v, out_do.at[wid])
    pltpu.sync_copy(degi_v, out_di.at[wid])


def _deg_call(src_p, dst_p):
    k = pl.kernel(
        _deg_body,
        out_type=(
            jax.ShapeDtypeStruct((32, _NP), jnp.float32),
            jax.ShapeDtypeStruct((32, _NP), jnp.float32),
        ),
        mesh=plsc.VectorSubcoreMesh(core_axis_name="c", subcore_axis_name="s"),
        scratch_types=[
            pltpu.VMEM((_T32,), jnp.int32),
            pltpu.VMEM((_T32,), jnp.int32),
            pltpu.VMEM((_NP,), jnp.float32),
            pltpu.VMEM((_NP,), jnp.float32),
        ],
        compiler_params=pltpu.CompilerParams(needs_layout_passes=False),
    )
    return k(src_p, dst_p)


def _agg_body(z0_hbm, z1_hbm, pk3_hbm, out_hbm,
              pk_v, rowsa_v, rowsb_v, sia_v, sib_v, dia_v, dib_v,
              agg_sp, ga, gb, sa, sb):
    c = lax.axis_index("c")
    s = lax.axis_index("s")
    zero16 = jnp.zeros((16,), jnp.float32)

    def zloop(r, carry):
        for j in range(_HH // 16):
            rowsa_v[r, pl.ds(j * 16, 16)] = zero16
        return carry

    lax.fori_loop(0, _CHUNK, zloop, 0)
    myrows = _NP // _NTILES
    rbase = s * myrows
    for kk in range(myrows // _CHUNK):
        pltpu.sync_copy(rowsa_v, agg_sp.at[pl.ds(rbase + kk * _CHUNK,
                                                 _CHUNK)])
    plsc.subcore_barrier()

    pltpu.sync_copy(pk3_hbm.at[s], pk_v)
    mask16 = jnp.full((16,), 0xFFFF, jnp.int32)

    def unpack(j, buf, lo):
        for t in range(_CHUNK // 16):
            w = pk_v[j, pl.ds(t * 16, 16)]
            if lo:
                buf[pl.ds(t * 16, 16)] = lax.bitwise_and(w, mask16)
            else:
                buf[pl.ds(t * 16, 16)] = lax.shift_right_logical(w, 16)

    def chunk_pair(z_hbm, g):
        ja = 2 * g
        jb = 2 * g + 1
        unpack(ja, sia_v, False)
        da = pltpu.async_copy(z_hbm.at[sia_v], rowsa_v, ga)
        unpack(jb, sib_v, False)
        db = pltpu.async_copy(z_hbm.at[sib_v], rowsb_v, gb)
        unpack(ja, dia_v, True)
        da.wait()
        dsa = pltpu.async_copy(rowsa_v, agg_sp.at[dia_v], sa, add=True)
        unpack(jb, dib_v, True)
        db.wait()
        dsb = pltpu.async_copy(rowsb_v, agg_sp.at[dib_v], sb, add=True)
        dsa.wait()
        dsb.wait()

    def eloop(g, carry):
        @pl.when(c == 0)
        def _():
            chunk_pair(z0_hbm, g)

        @pl.when(c == 1)
        def _():
            chunk_pair(z1_hbm, g)

        return carry

    lax.fori_loop(0, _NCHUNK // 2, eloop, 0)
    plsc.subcore_barrier()
    pltpu.sync_copy(agg_sp.at[pl.ds(rbase, myrows)],
                    out_hbm.at[c, pl.ds(rbase, myrows)])


def _agg_call(z0, z1, pk3):
    k = pl.kernel(
        _agg_body,
        out_type=jax.ShapeDtypeStruct((2, _NP, _HH), jnp.float32),
        mesh=plsc.VectorSubcoreMesh(core_axis_name="c", subcore_axis_name="s"),
        scratch_types=[
            pltpu.VMEM((_NCHUNK, _CHUNK), jnp.int32),
            pltpu.VMEM((_CHUNK, _HH), jnp.float32),
            pltpu.VMEM((_CHUNK, _HH), jnp.float32),
            pltpu.VMEM((_CHUNK,), jnp.int32),
            pltpu.VMEM((_CHUNK,), jnp.int32),
            pltpu.VMEM((_CHUNK,), jnp.int32),
            pltpu.VMEM((_CHUNK,), jnp.int32),
            pltpu.VMEM_SHARED((_NP, _HH), jnp.float32),
            pltpu.SemaphoreType.DMA,
            pltpu.SemaphoreType.DMA,
            pltpu.SemaphoreType.DMA,
            pltpu.SemaphoreType.DMA,
        ],
        compiler_params=pltpu.CompilerParams(needs_layout_passes=False),
    )
    return k(z0, z1, pk3)


def _row_mask():
    rows = (lax.broadcasted_iota(jnp.int32, (_BM, 1), 0)
            + pl.program_id(0) * _BM)
    return (rows < _N).astype(jnp.float32)


def _stage0_body(dpo_ref, dpi_ref, feats_ref, inW_ref, inb_ref, W1_ref,
                 no_ref, ni_ref, h0_ref, z0_ref, z1_ref):
    dgo = jnp.sum(dpo_ref[...], axis=1, keepdims=True)
    dgi = jnp.sum(dpi_ref[...], axis=1, keepdims=True)
    no = jnp.where(dgo > 0, lax.rsqrt(dgo), 0.0)
    ni = jnp.where(dgi > 0, lax.rsqrt(dgi), 0.0)
    no_ref[...] = no
    ni_ref[...] = ni
    h0 = jnp.dot(feats_ref[...], inW_ref[...],
                 preferred_element_type=jnp.float32) + inb_ref[...]
    h0_ref[...] = h0
    z = jnp.dot(h0 * no * _row_mask(), W1_ref[...],
                preferred_element_type=jnp.float32)
    z0_ref[...] = z[:, :_HH]
    z1_ref[...] = z[:, _HH:]


def _stage0_call(dpoT, dpiT, feats, inW, inb, W1):
    return pl.pallas_call(
        _stage0_body,
        grid=(_GRID,),
        in_specs=[
            pl.BlockSpec((_BM, 32), lambda m: (m, 0)),
            pl.BlockSpec((_BM, 32), lambda m: (m, 0)),
            pl.BlockSpec((_BM, _HH), lambda m: (m, 0)),
            pl.BlockSpec((_HH, _H), lambda m: (0, 0)),
            pl.BlockSpec((1, _H), lambda m: (0, 0)),
            pl.BlockSpec((_H, _H), lambda m: (0, 0)),
        ],
        out_specs=[
            pl.BlockSpec((_BM, 1), lambda m: (m, 0)),
            pl.BlockSpec((_BM, 1), lambda m: (m, 0)),
            pl.BlockSpec((_BM, _H), lambda m: (m, 0)),
            pl.BlockSpec((_BM, _HH), lambda m: (m, 0)),
            pl.BlockSpec((_BM, _HH), lambda m: (m, 0)),
        ],
        out_shape=[
            jax.ShapeDtypeStruct((_NP, 1), jnp.float32),
            jax.ShapeDtypeStruct((_NP, 1), jnp.float32),
            jax.ShapeDtypeStruct((_NP, _H), jnp.float32),
            jax.ShapeDtypeStruct((_NP, _HH), jnp.float32),
            jax.ShapeDtypeStruct((_NP, _HH), jnp.float32),
        ],
    )(dpoT, dpiT, feats, inW, inb, W1)


def _mid1_body(agg_ref, ni_ref, no_ref, s_ref, b_ref, W2_ref, z0_ref, z1_ref):
    a = jnp.concatenate([agg_ref[0], agg_ref[1]], axis=1) * ni_ref[...]
    y = jnp.maximum(a * s_ref[...] + b_ref[...], 0.0)
    z = jnp.dot(y * no_ref[...] * _row_mask(), W2_ref[...],
                preferred_element_type=jnp.float32)
    z0_ref[...] = z[:, :_HH]
    z1_ref[...] = z[:, _HH:]


def _mid1_call(agg, ni, no, sv, bv, W2):
    return pl.pallas_call(
        _mid1_body,
        grid=(_GRID,),
        in_specs=[
            pl.BlockSpec((2, _BM, _HH), lambda m: (0, m, 0)),
            pl.BlockSpec((_BM, 1), lambda m: (m, 0)),
            pl.BlockSpec((_BM, 1), lambda m: (m, 0)),
            pl.BlockSpec((1, _H), lambda m: (0, 0)),
            pl.BlockSpec((1, _H), lambda m: (0, 0)),
            pl.BlockSpec((_H, _H), lambda m: (0, 0)),
        ],
        out_specs=[
            pl.BlockSpec((_BM, _HH), lambda m: (m, 0)),
            pl.BlockSpec((_BM, _HH), lambda m: (m, 0)),
        ],
        out_shape=[
            jax.ShapeDtypeStruct((_NP, _HH), jnp.float32),
            jax.ShapeDtypeStruct((_NP, _HH), jnp.float32),
        ],
    )(agg, ni, no, sv, bv, W2)


def _mid2_body(agg_ref, ni_ref, no_ref, s_ref, b_ref, ffW_ref, ffb_ref,
               hin_ref, Wn_ref, h_ref, z0_ref, z1_ref):
    a = jnp.concatenate([agg_ref[0], agg_ref[1]], axis=1) * ni_ref[...]
    y = jnp.maximum(a * s_ref[...] + b_ref[...], 0.0)
    h = (jnp.dot(y, ffW_ref[...], preferred_element_type=jnp.float32)
         + ffb_ref[...] + hin_ref[...])
    h_ref[...] = h
    z = jnp.dot(h * no_ref[...] * _row_mask(), Wn_ref[...],
                preferred_element_type=jnp.float32)
    z0_ref[...] = z[:, :_HH]
    z1_ref[...] = z[:, _HH:]


def _mid2_call(agg, ni, no, sv, bv, ffW, ffb, hin, Wn):
    return pl.pallas_call(
        _mid2_body,
        grid=(_GRID,),
        in_specs=[
            pl.BlockSpec((2, _BM, _HH), lambda m: (0, m, 0)),
            pl.BlockSpec((_BM, 1), lambda m: (m, 0)),
            pl.BlockSpec((_BM, 1), lambda m: (m, 0)),
            pl.BlockSpec((1, _H), lambda m: (0, 0)),
            pl.BlockSpec((1, _H), lambda m: (0, 0)),
            pl.BlockSpec((_H, _H), lambda m: (0, 0)),
            pl.BlockSpec((1, _H), lambda m: (0, 0)),
            pl.BlockSpec((_BM, _H), lambda m: (m, 0)),
            pl.BlockSpec((_H, _H), lambda m: (0, 0)),
        ],
        out_specs=[
            pl.BlockSpec((_BM, _H), lambda m: (m, 0)),
            pl.BlockSpec((_BM, _HH), lambda m: (m, 0)),
            pl.BlockSpec((_BM, _HH), lambda m: (m, 0)),
        ],
        out_shape=[
            jax.ShapeDtypeStruct((_NP, _H), jnp.float32),
            jax.ShapeDtypeStruct((_NP, _HH), jnp.float32),
            jax.ShapeDtypeStruct((_NP, _HH), jnp.float32),
        ],
    )(agg, ni, no, sv, bv, ffW, ffb, hin, Wn)


def _fin_body(agg_ref, ni_ref, s_ref, b_ref, ffW_ref, ffb_ref, hin_ref,
              o1W_ref, o1b_ref, o2W_ref, o2b_ref, out_ref):
    a = jnp.concatenate([agg_ref[0], agg_ref[1]], axis=1) * ni_ref[...]
    y = jnp.maximum(a * s_ref[...] + b_ref[...], 0.0)
    h = (jnp.dot(y, ffW_ref[...], preferred_element_type=jnp.float32)
         + ffb_ref[...] + hin_ref[...])
    t = jnp.maximum(jnp.dot(h, o1W_ref[...],
                            preferred_element_type=jnp.float32)
                    + o1b_ref[...], 0.0)
    out_ref[...] = jnp.dot(t, o2W_ref[...],
                           preferred_element_type=jnp.float32) + o2b_ref[...]


def _fin_call(agg, ni, sv, bv, ffW, ffb, hin, o1W, o1b, o2W, o2b):
    return pl.pallas_call(
        _fin_body,
        grid=(_GRID,),
        in_specs=[
            pl.BlockSpec((2, _BM, _HH), lambda m: (0, m, 0)),
            pl.BlockSpec((_BM, 1), lambda m: (m, 0)),
            pl.BlockSpec((1, _H), lambda m: (0, 0)),
            pl.BlockSpec((1, _H), lambda m: (0, 0)),
            pl.BlockSpec((_H, _H), lambda m: (0, 0)),
            pl.BlockSpec((1, _H), lambda m: (0, 0)),
            pl.BlockSpec((_BM, _H), lambda m: (m, 0)),
            pl.BlockSpec((_H, _H), lambda m: (0, 0)),
            pl.BlockSpec((1, _H), lambda m: (0, 0)),
            pl.BlockSpec((_H, _HH), lambda m: (0, 0)),
            pl.BlockSpec((1, _HH), lambda m: (0, 0)),
        ],
        out_specs=pl.BlockSpec((_BM, _HH), lambda m: (m, 0)),
        out_shape=jax.ShapeDtypeStruct((_NP, _HH), jnp.float32),
    )(agg, ni, sv, bv, ffW, ffb, hin, o1W, o1b, o2W, o2b)


def kernel(node_2d_features, lap_pe, edge_index, in_W, in_b, conv1_W, conv1_b,
           bn1_g, bn1_b, conv2_W, conv2_b, bn2_g, bn2_b, ff_W, ff_b,
           out1_W, out1_b, out2_W, out2_b):
    src = edge_index[0].astype(jnp.int32)
    dst = edge_index[1].astype(jnp.int32)
    pad = _EP - _E
    src_p = jnp.concatenate([src, jnp.full((pad,), _DUMMY, jnp.int32)])
    dst_p = jnp.concatenate([dst, jnp.full((pad,), _DUMMY, jnp.int32)])
    pk3 = ((src_p << 16) | dst_p).reshape(_NTILES, _NCHUNK, _CHUNK)

    feats = jnp.concatenate([node_2d_features, lap_pe], axis=1)
    feats = jnp.pad(feats, ((0, _NP - _N), (0, _HH - (2 + _K))))
    inW_p = jnp.pad(in_W, ((0, _HH - (2 + _K)), (0, 0)))
    inb = in_b[None, :]
    rb = 1.0 / jnp.sqrt(jnp.float32(1.0 + 1e-5))
    s1 = bn1_g * rb
    b1m = conv1_b * s1 + bn1_b
    s2 = bn2_g * rb
    b2m = conv2_b * s2 + bn2_b
    o2W_p = jnp.pad(out2_W, ((0, 0), (0, _HH - 3)))
    o2b_p = jnp.pad(out2_b, (0, _HH - 3))[None, :]

    dpo, dpi = _deg_call(src_p, dst_p)
    no, ni, h, z0, z1 = _stage0_call(dpo.T, dpi.T, feats, inW_p, inb,
                                     conv1_W[0])
    for i in range(_NM):
        agg = _agg_call(z0, z1, pk3)
        z0, z1 = _mid1_call(agg, ni, no, s1[i][None], b1m[i][None],
                            conv2_W[i])
        agg = _agg_call(z0, z1, pk3)
        if i < _NM - 1:
            h, z0, z1 = _mid2_call(agg, ni, no, s2[i][None], b2m[i][None],
                                   ff_W[i], ff_b[i][None], h, conv1_W[i + 1])
        else:
            out = _fin_call(agg, ni, s2[i][None], b2m[i][None], ff_W[i],
                            ff_b[i][None], h, out1_W, out1_b[None],
                            o2W_p, o2b_p)
    return out[:_N, :3]


# speedup vs baseline: 1.3569x; 1.3569x over previous
import jax
import jax.numpy as jnp
from jax import lax
from jax.experimental import pallas as pl
from jax.experimental.pallas import tpu as pltpu
from jax.experimental.pallas import tpu_sc as plsc

_N = 10000
_NP = 10240
_H = 256
_HH = 128
_E = 160000
_K = 20
_NM = 8
_NTILES = 16
_NCORES = 2
_CHUNK = 128
_NCHUNK = 79
_EPT = _NCHUNK * _CHUNK
_EP = _EPT * _NTILES
_T32 = _EP // 32
_DUMMY = _N
_BM = 1024
_GRID = _NP // _BM


def _deg_body(src_hbm, dst_hbm, out_do, out_di, src_v, dst_v, dego_v, degi_v):
    c = lax.axis_index("c")
    s = lax.axis_index("s")
    wid = s * _NCORES + c
    zero16 = jnp.zeros((16,), jnp.float32)

    def zloop(i, carry):
        dego_v[pl.ds(i * 16, 16)] = zero16
        degi_v[pl.ds(i * 16, 16)] = zero16
        return carry

    lax.fori_loop(0, _NP // 16, zloop, 0)

    base = wid * _T32
    pltpu.sync_copy(src_hbm.at[pl.ds(base, _T32)], src_v)
    pltpu.sync_copy(dst_hbm.at[pl.ds(base, _T32)], dst_v)
    ones16 = jnp.ones((16,), jnp.float32)

    def eloop(i, carry):
        si = src_v[pl.ds(i * 16, 16)]
        plsc.addupdate_scatter(dego_v, [si], ones16)
        di = dst_v[pl.ds(i * 16, 16)]
        plsc.addupdate_scatter(degi_v, [di], ones16)
        return carry

    lax.fori_loop(0, _T32 // 16, eloop, 0)
    pltpu.sync_copy(dego_v, out_do.at[wid])
    pltpu.sync_copy(degi_v, out_di.at[wid])


def _deg_call(src_p, dst_p):
    k = pl.kernel(
        _deg_body,
        out_type=(
            jax.ShapeDtypeStruct((32, _NP), jnp.float32),
            jax.ShapeDtypeStruct((32, _NP), jnp.float32),
        ),
        mesh=plsc.VectorSubcoreMesh(core_axis_name="c", subcore_axis_name="s"),
        scratch_types=[
            pltpu.VMEM((_T32,), jnp.int32),
            pltpu.VMEM((_T32,), jnp.int32),
            pltpu.VMEM((_NP,), jnp.float32),
            pltpu.VMEM((_NP,), jnp.float32),
        ],
        compiler_params=pltpu.CompilerParams(needs_layout_passes=False),
    )
    return k(src_p, dst_p)


def _agg_body(z0_hbm, z1_hbm, src3_hbm, dst3_hbm, out_hbm,
              sall_v, dall_v, rows_v, agg_sp, sem):
    c = lax.axis_index("c")
    s = lax.axis_index("s")
    zero16 = jnp.zeros((16,), jnp.float32)

    def zloop(r, carry):
        for j in range(_HH // 16):
            rows_v[r, pl.ds(j * 16, 16)] = zero16
        return carry

    lax.fori_loop(0, _CHUNK, zloop, 0)
    myrows = _NP // _NTILES
    r0 = s * myrows
    for kk in range(myrows // _CHUNK):
        pltpu.sync_copy(rows_v, agg_sp.at[pl.ds(r0 + kk * _CHUNK, _CHUNK)])
    plsc.subcore_barrier()

    pltpu.sync_copy(src3_hbm.at[s], sall_v)
    pltpu.sync_copy(dst3_hbm.at[s], dall_v)

    def eloop(j, carry):
        idx = sall_v.at[j]

        @pl.when(c == 0)
        def _():
            pltpu.async_copy(z0_hbm.at[idx], rows_v, sem).wait()

        @pl.when(c == 1)
        def _():
            pltpu.async_copy(z1_hbm.at[idx], rows_v, sem).wait()

        pltpu.sync_copy(rows_v, agg_sp.at[dall_v.at[j]], add=True)
        return carry

    lax.fori_loop(0, _NCHUNK, eloop, 0)
    plsc.subcore_barrier()
    pltpu.sync_copy(agg_sp.at[pl.ds(r0, myrows)],
                    out_hbm.at[c, pl.ds(r0, myrows)])


def _agg_call(z0, z1, src3, dst3):
    k = pl.kernel(
        _agg_body,
        out_type=jax.ShapeDtypeStruct((2, _NP, _HH), jnp.float32),
        mesh=plsc.VectorSubcoreMesh(core_axis_name="c", subcore_axis_name="s"),
        scratch_types=[
            pltpu.VMEM((_NCHUNK, _CHUNK), jnp.int32),
            pltpu.VMEM((_NCHUNK, _CHUNK), jnp.int32),
            pltpu.VMEM((_CHUNK, _HH), jnp.float32),
            pltpu.VMEM_SHARED((_NP, _HH), jnp.float32),
            pltpu.SemaphoreType.DMA,
        ],
        compiler_params=pltpu.CompilerParams(needs_layout_passes=False),
    )
    return k(z0, z1, src3, dst3)


def _row_mask():
    rows = (lax.broadcasted_iota(jnp.int32, (_BM, 1), 0)
            + pl.program_id(0) * _BM)
    return (rows < _N).astype(jnp.float32)


def _stage0_body(dpo_ref, dpi_ref, feats_ref, inW_ref, inb_ref, W1_ref,
                 no_ref, ni_ref, h0_ref, z0_ref, z1_ref):
    dgo = jnp.sum(dpo_ref[...], axis=1, keepdims=True)
    dgi = jnp.sum(dpi_ref[...], axis=1, keepdims=True)
    no = jnp.where(dgo > 0, lax.rsqrt(dgo), 0.0)
    ni = jnp.where(dgi > 0, lax.rsqrt(dgi), 0.0)
    no_ref[...] = no
    ni_ref[...] = ni
    h0 = jnp.dot(feats_ref[...], inW_ref[...],
                 preferred_element_type=jnp.float32) + inb_ref[...]
    h0_ref[...] = h0
    z = jnp.dot(h0 * no * _row_mask(), W1_ref[...],
                preferred_element_type=jnp.float32)
    z0_ref[...] = z[:, :_HH]
    z1_ref[...] = z[:, _HH:]


def _stage0_call(dpoT, dpiT, feats, inW, inb, W1):
    return pl.pallas_call(
        _stage0_body,
        grid=(_GRID,),
        in_specs=[
            pl.BlockSpec((_BM, 32), lambda m: (m, 0)),
            pl.BlockSpec((_BM, 32), lambda m: (m, 0)),
            pl.BlockSpec((_BM, _HH), lambda m: (m, 0)),
            pl.BlockSpec((_HH, _H), lambda m: (0, 0)),
            pl.BlockSpec((1, _H), lambda m: (0, 0)),
            pl.BlockSpec((_H, _H), lambda m: (0, 0)),
        ],
        out_specs=[
            pl.BlockSpec((_BM, 1), lambda m: (m, 0)),
            pl.BlockSpec((_BM, 1), lambda m: (m, 0)),
            pl.BlockSpec((_BM, _H), lambda m: (m, 0)),
            pl.BlockSpec((_BM, _HH), lambda m: (m, 0)),
            pl.BlockSpec((_BM, _HH), lambda m: (m, 0)),
        ],
        out_shape=[
            jax.ShapeDtypeStruct((_NP, 1), jnp.float32),
            jax.ShapeDtypeStruct((_NP, 1), jnp.float32),
            jax.ShapeDtypeStruct((_NP, _H), jnp.float32),
            jax.ShapeDtypeStruct((_NP, _HH), jnp.float32),
            jax.ShapeDtypeStruct((_NP, _HH), jnp.float32),
        ],
    )(dpoT, dpiT, feats, inW, inb, W1)


def _mid1_body(agg_ref, ni_ref, no_ref, s_ref, b_ref, W2_ref, z0_ref, z1_ref):
    a = jnp.concatenate([agg_ref[0], agg_ref[1]], axis=1) * ni_ref[...]
    y = jnp.maximum(a * s_ref[...] + b_ref[...], 0.0)
    z = jnp.dot(y * no_ref[...] * _row_mask(), W2_ref[...],
                preferred_element_type=jnp.float32)
    z0_ref[...] = z[:, :_HH]
    z1_ref[...] = z[:, _HH:]


def _mid1_call(agg, ni, no, sv, bv, W2):
    return pl.pallas_call(
        _mid1_body,
        grid=(_GRID,),
        in_specs=[
            pl.BlockSpec((2, _BM, _HH), lambda m: (0, m, 0)),
            pl.BlockSpec((_BM, 1), lambda m: (m, 0)),
            pl.BlockSpec((_BM, 1), lambda m: (m, 0)),
            pl.BlockSpec((1, _H), lambda m: (0, 0)),
            pl.BlockSpec((1, _H), lambda m: (0, 0)),
            pl.BlockSpec((_H, _H), lambda m: (0, 0)),
        ],
        out_specs=[
            pl.BlockSpec((_BM, _HH), lambda m: (m, 0)),
            pl.BlockSpec((_BM, _HH), lambda m: (m, 0)),
        ],
        out_shape=[
            jax.ShapeDtypeStruct((_NP, _HH), jnp.float32),
            jax.ShapeDtypeStruct((_NP, _HH), jnp.float32),
        ],
    )(agg, ni, no, sv, bv, W2)


def _mid2_body(agg_ref, ni_ref, no_ref, s_ref, b_ref, ffW_ref, ffb_ref,
               hin_ref, Wn_ref, h_ref, z0_ref, z1_ref):
    a = jnp.concatenate([agg_ref[0], agg_ref[1]], axis=1) * ni_ref[...]
    y = jnp.maximum(a * s_ref[...] + b_ref[...], 0.0)
    h = (jnp.dot(y, ffW_ref[...], preferred_element_type=jnp.float32)
         + ffb_ref[...] + hin_ref[...])
    h_ref[...] = h
    z = jnp.dot(h * no_ref[...] * _row_mask(), Wn_ref[...],
                preferred_element_type=jnp.float32)
    z0_ref[...] = z[:, :_HH]
    z1_ref[...] = z[:, _HH:]


def _mid2_call(agg, ni, no, sv, bv, ffW, ffb, hin, Wn):
    return pl.pallas_call(
        _mid2_body,
        grid=(_GRID,),
        in_specs=[
            pl.BlockSpec((2, _BM, _HH), lambda m: (0, m, 0)),
            pl.BlockSpec((_BM, 1), lambda m: (m, 0)),
            pl.BlockSpec((_BM, 1), lambda m: (m, 0)),
            pl.BlockSpec((1, _H), lambda m: (0, 0)),
            pl.BlockSpec((1, _H), lambda m: (0, 0)),
            pl.BlockSpec((_H, _H), lambda m: (0, 0)),
            pl.BlockSpec((1, _H), lambda m: (0, 0)),
            pl.BlockSpec((_BM, _H), lambda m: (m, 0)),
            pl.BlockSpec((_H, _H), lambda m: (0, 0)),
        ],
        out_specs=[
            pl.BlockSpec((_BM, _H), lambda m: (m, 0)),
            pl.BlockSpec((_BM, _HH), lambda m: (m, 0)),
            pl.BlockSpec((_BM, _HH), lambda m: (m, 0)),
        ],
        out_shape=[
            jax.ShapeDtypeStruct((_NP, _H), jnp.float32),
            jax.ShapeDtypeStruct((_NP, _HH), jnp.float32),
            jax.ShapeDtypeStruct((_NP, _HH), jnp.float32),
        ],
    )(agg, ni, no, sv, bv, ffW, ffb, hin, Wn)


def _fin_body(agg_ref, ni_ref, s_ref, b_ref, ffW_ref, ffb_ref, hin_ref,
              o1W_ref, o1b_ref, o2W_ref, o2b_ref, out_ref):
    a = jnp.concatenate([agg_ref[0], agg_ref[1]], axis=1) * ni_ref[...]
    y = jnp.maximum(a * s_ref[...] + b_ref[...], 0.0)
    h = (jnp.dot(y, ffW_ref[...], preferred_element_type=jnp.float32)
         + ffb_ref[...] + hin_ref[...])
    t = jnp.maximum(jnp.dot(h, o1W_ref[...],
                            preferred_element_type=jnp.float32)
                    + o1b_ref[...], 0.0)
    out_ref[...] = jnp.dot(t, o2W_ref[...],
                           preferred_element_type=jnp.float32) + o2b_ref[...]


def _fin_call(agg, ni, sv, bv, ffW, ffb, hin, o1W, o1b, o2W, o2b):
    return pl.pallas_call(
        _fin_body,
        grid=(_GRID,),
        in_specs=[
            pl.BlockSpec((2, _BM, _HH), lambda m: (0, m, 0)),
            pl.BlockSpec((_BM, 1), lambda m: (m, 0)),
            pl.BlockSpec((1, _H), lambda m: (0, 0)),
            pl.BlockSpec((1, _H), lambda m: (0, 0)),
            pl.BlockSpec((_H, _H), lambda m: (0, 0)),
            pl.BlockSpec((1, _H), lambda m: (0, 0)),
            pl.BlockSpec((_BM, _H), lambda m: (m, 0)),
            pl.BlockSpec((_H, _H), lambda m: (0, 0)),
            pl.BlockSpec((1, _H), lambda m: (0, 0)),
            pl.BlockSpec((_H, _HH), lambda m: (0, 0)),
            pl.BlockSpec((1, _HH), lambda m: (0, 0)),
        ],
        out_specs=pl.BlockSpec((_BM, _HH), lambda m: (m, 0)),
        out_shape=jax.ShapeDtypeStruct((_NP, _HH), jnp.float32),
    )(agg, ni, sv, bv, ffW, ffb, hin, o1W, o1b, o2W, o2b)


def kernel(node_2d_features, lap_pe, edge_index, in_W, in_b, conv1_W, conv1_b,
           bn1_g, bn1_b, conv2_W, conv2_b, bn2_g, bn2_b, ff_W, ff_b,
           out1_W, out1_b, out2_W, out2_b):
    src = edge_index[0].astype(jnp.int32)
    dst = edge_index[1].astype(jnp.int32)
    pad = _EP - _E
    src_p = jnp.concatenate([src, jnp.full((pad,), _DUMMY, jnp.int32)])
    dst_p = jnp.concatenate([dst, jnp.full((pad,), _DUMMY, jnp.int32)])
    src3 = src_p.reshape(_NTILES, _NCHUNK, _CHUNK)
    dst3 = dst_p.reshape(_NTILES, _NCHUNK, _CHUNK)

    feats = jnp.concatenate([node_2d_features, lap_pe], axis=1)
    feats = jnp.pad(feats, ((0, _NP - _N), (0, _HH - (2 + _K))))
    inW_p = jnp.pad(in_W, ((0, _HH - (2 + _K)), (0, 0)))
    inb = in_b[None, :]
    rb = 1.0 / jnp.sqrt(jnp.float32(1.0 + 1e-5))
    s1 = bn1_g * rb
    b1m = conv1_b * s1 + bn1_b
    s2 = bn2_g * rb
    b2m = conv2_b * s2 + bn2_b
    o2W_p = jnp.pad(out2_W, ((0, 0), (0, _HH - 3)))
    o2b_p = jnp.pad(out2_b, (0, _HH - 3))[None, :]

    dpo, dpi = _deg_call(src_p, dst_p)
    no, ni, h, z0, z1 = _stage0_call(dpo.T, dpi.T, feats, inW_p, inb,
                                     conv1_W[0])
    for i in range(_NM):
        agg = _agg_call(z0, z1, src3, dst3)
        z0, z1 = _mid1_call(agg, ni, no, s1[i][None], b1m[i][None],
                            conv2_W[i])
        agg = _agg_call(z0, z1, src3, dst3)
        if i < _NM - 1:
            h, z0, z1 = _mid2_call(agg, ni, no, s2[i][None], b2m[i][None],
                                   ff_W[i], ff_b[i][None], h, conv1_W[i + 1])
        else:
            out = _fin_call(agg, ni, s2[i][None], b2m[i][None], ff_W[i],
                            ff_b[i][None], h, out1_W, out1_b[None],
                            o2W_p, o2b_p)
    return out[:_N, :3]
